# baseline (speedup 1.0000x reference)
import jax
import jax.numpy as jnp
from jax.experimental import pallas as pl
from jax.experimental.pallas import tpu as pltpu

_ROWS = 131072
_COLS = 257
_SLABS = 8
_SLAB = _ROWS // _SLABS


def _dma_kernel(in_hbm, out_hbm, sems):
    def issue(slab):
        rows = pl.ds(slab * _SLAB, _SLAB)
        c0 = pltpu.make_async_copy(
            in_hbm.at[rows, pl.ds(128, 128)], out_hbm.at[rows, pl.ds(0, 128)],
            sems.at[slab, 0])
        c1 = pltpu.make_async_copy(
            in_hbm.at[rows, pl.ds(0, 128)], out_hbm.at[rows, pl.ds(128, 128)],
            sems.at[slab, 1])
        c2 = pltpu.make_async_copy(
            in_hbm.at[rows, pl.ds(256, 1)], out_hbm.at[rows, pl.ds(256, 1)],
            sems.at[slab, 2])
        return c0, c1, c2

    copies = []
    for s in range(_SLABS):
        cs = issue(s)
        for c in cs:
            c.start()
        copies.append(cs)
    for cs in copies:
        for c in cs:
            c.wait()


def kernel(tensor, list_ind):
    del list_ind
    return pl.pallas_call(
        _dma_kernel,
        in_specs=[pl.BlockSpec(memory_space=pl.ANY)],
        out_specs=pl.BlockSpec(memory_space=pl.ANY),
        out_shape=jax.ShapeDtypeStruct((_ROWS, _COLS), tensor.dtype),
        scratch_shapes=[pltpu.SemaphoreType.DMA((_SLABS, 3))],
    )(tensor)


# speedup vs baseline: 14.8632x; 14.8632x over previous
import jax
import jax.numpy as jnp
from jax.experimental import pallas as pl
from jax.experimental.pallas import tpu as pltpu

_ROWS = 131072
_COLS = 257
_SLAB = 4096
_N = _ROWS // _SLAB
_K = 4
_CH = _SLAB // _K


def _pipeline_kernel(in_hbm, out_hbm, in_buf, out_buf, in_sems, out_sems):
    i = pl.program_id(0)
    slot = jax.lax.rem(i, 2)
    nslot = jax.lax.rem(i + 1, 2)

    def in_copy(slab_idx, slot_, k):
        return pltpu.make_async_copy(
            in_hbm.at[pl.ds(slab_idx * _SLAB + k * _CH, _CH), :],
            in_buf.at[slot_, pl.ds(k * _CH, _CH), :],
            in_sems.at[slot_, k])

    def out_copy(slab_idx, slot_, k):
        return pltpu.make_async_copy(
            out_buf.at[slot_, pl.ds(k * _CH, _CH), :],
            out_hbm.at[pl.ds(slab_idx * _SLAB + k * _CH, _CH), :],
            out_sems.at[slot_, k])

    @pl.when(i == 0)
    def _():
        for k in range(_K):
            in_copy(i, slot, k).start()

    @pl.when(i + 1 < _N)
    def _():
        for k in range(_K):
            in_copy(i + 1, nslot, k).start()

    for k in range(_K):
        in_copy(i, slot, k).wait()

    @pl.when(i >= 2)
    def _():
        for k in range(_K):
            out_copy(i - 2, slot, k).wait()

    out_buf[slot, :, 0:128] = in_buf[slot, :, 128:256]
    out_buf[slot, :, 128:256] = in_buf[slot, :, 0:128]
    out_buf[slot, :, 256:257] = in_buf[slot, :, 256:257]

    for k in range(_K):
        out_copy(i, slot, k).start()

    @pl.when(i == _N - 1)
    def _():
        for k in range(_K):
            out_copy(i - 1, nslot, k).wait()
        for k in range(_K):
            out_copy(i, slot, k).wait()


def kernel(tensor, list_ind):
    del list_ind
    return pl.pallas_call(
        _pipeline_kernel,
        grid=(_N,),
        in_specs=[pl.BlockSpec(memory_space=pl.ANY)],
        out_specs=pl.BlockSpec(memory_space=pl.ANY),
        out_shape=jax.ShapeDtypeStruct((_ROWS, _COLS), tensor.dtype),
        scratch_shapes=[
            pltpu.VMEM((2, _SLAB, _COLS), jnp.float32),
            pltpu.VMEM((2, _SLAB, _COLS), jnp.float32),
            pltpu.SemaphoreType.DMA((2, _K)),
            pltpu.SemaphoreType.DMA((2, _K)),
        ],
    )(tensor)
